# Initial kernel scaffold; baseline (speedup 1.0000x reference)
import functools
import jax
import jax.numpy as jnp
from jax import lax
from jax.experimental import pallas as pl
from jax.experimental.pallas import tpu as pltpu
from jax.experimental.pallas import tpu_sc as plsc

N_NODES = 10000
N_EDGES = 320000
HIDDEN = 128
NUM_LAYERS = 3
N_GRAPHS = 64
N_PAIRS = 4096

NC = 2
NS = 16
NW = NC * NS

CHUNK = 80
EDGES_PER_TILE = N_EDGES // NW
NCH = EDGES_PER_TILE // CHUNK
ROWS_PER_TILE = N_NODES // NS

_mesh = plsc.VectorSubcoreMesh(core_axis_name="c", subcore_axis_name="s")


@functools.partial(
    pl.kernel,
    out_type=jax.ShapeDtypeStruct((NC, N_NODES, HIDDEN), jnp.float32),
    mesh=_mesh,
    scratch_types=[
        pltpu.VMEM_SHARED((N_NODES, HIDDEN), jnp.float32),
        pltpu.VMEM((NCH, CHUNK), jnp.int32),
        pltpu.VMEM((NCH, CHUNK), jnp.int32),
        pltpu.VMEM((CHUNK, HIDDEN), jnp.float32),
        pltpu.VMEM((CHUNK, HIDDEN), jnp.float32),
        pltpu.SemaphoreType.DMA,
        pltpu.SemaphoreType.DMA,
    ],
)
def _edge_agg(h_hbm, src_hbm, dst_hbm, zeros_hbm, out_hbm,
              acc, idx_s, idx_d, rows0, rows1, sem0, sem1):
    c = lax.axis_index("c")
    s = lax.axis_index("s")
    wid = s * NC + c

    pltpu.sync_copy(zeros_hbm, acc.at[pl.ds(s * ROWS_PER_TILE, ROWS_PER_TILE)])
    pltpu.sync_copy(src_hbm.at[pl.ds(wid * NCH, NCH)], idx_s)
    pltpu.sync_copy(dst_hbm.at[pl.ds(wid * NCH, NCH)], idx_d)
    plsc.subcore_barrier()

    pltpu.async_copy(h_hbm.at[idx_s.at[0]], rows0, sem0)

    def body(j, _):
        @pl.when(j % 2 == 0)
        def _even():
            pltpu.make_async_copy(h_hbm.at[idx_s.at[j]], rows0, sem0).wait()

            @pl.when(j + 1 < NCH)
            def _():
                pltpu.async_copy(h_hbm.at[idx_s.at[j + 1]], rows1, sem1)
            pltpu.sync_copy(rows0, acc.at[idx_d.at[j]], add=True)

        @pl.when(j % 2 == 1)
        def _odd():
            pltpu.make_async_copy(h_hbm.at[idx_s.at[j]], rows1, sem1).wait()

            @pl.when(j + 1 < NCH)
            def _():
                pltpu.async_copy(h_hbm.at[idx_s.at[j + 1]], rows0, sem0)
            pltpu.sync_copy(rows1, acc.at[idx_d.at[j]], add=True)
        return _

    lax.fori_loop(0, NCH, body, None)
    plsc.subcore_barrier()

    pltpu.sync_copy(acc.at[pl.ds(s * ROWS_PER_TILE, ROWS_PER_TILE)],
                    out_hbm.at[c].at[pl.ds(s * ROWS_PER_TILE, ROWS_PER_TILE)])


PAIRS_PER_TILE = N_PAIRS // NW


@functools.partial(
    pl.kernel,
    out_type=[
        jax.ShapeDtypeStruct((N_PAIRS, HIDDEN), jnp.float32),
        jax.ShapeDtypeStruct((N_PAIRS, HIDDEN), jnp.float32),
    ],
    mesh=_mesh,
    scratch_types=[
        pltpu.VMEM((PAIRS_PER_TILE,), jnp.int32),
        pltpu.VMEM((PAIRS_PER_TILE, HIDDEN), jnp.float32),
        pltpu.SemaphoreType.DMA,
    ],
)
def _pair_gather(emb_hbm, p0_hbm, p1_hbm, ea_hbm, eb_hbm, idx_v, rows_v, sem):
    c = lax.axis_index("c")
    s = lax.axis_index("s")
    wid = s * NC + c
    base = wid * PAIRS_PER_TILE

    pltpu.sync_copy(p0_hbm.at[wid], idx_v)
    pltpu.async_copy(emb_hbm.at[idx_v], rows_v, sem).wait()
    pltpu.sync_copy(rows_v, ea_hbm.at[pl.ds(base, PAIRS_PER_TILE)])

    pltpu.sync_copy(p1_hbm.at[wid], idx_v)
    pltpu.async_copy(emb_hbm.at[idx_v], rows_v, sem).wait()
    pltpu.sync_copy(rows_v, eb_hbm.at[pl.ds(base, PAIRS_PER_TILE)])


def _layer_body(h_ref, a_ref, w1_ref, b1_ref, w2_ref, b2_ref, g_ref, be_ref, o_ref):
    m = h_ref[...] + a_ref[0] + a_ref[1]
    z = jnp.dot(m, w1_ref[...], preferred_element_type=jnp.float32) + b1_ref[...]
    z = jnp.maximum(z, 0.0)
    z = jnp.dot(z, w2_ref[...], preferred_element_type=jnp.float32) + b2_ref[...]
    mean = jnp.mean(z, axis=0, keepdims=True)
    cen = z - mean
    var = jnp.mean(cen * cen, axis=0, keepdims=True)
    z = cen * lax.rsqrt(var + 1e-5) * g_ref[...] + be_ref[...]
    o_ref[...] = jnp.maximum(z, 0.0)


_layer_call = pl.pallas_call(
    _layer_body,
    out_shape=jax.ShapeDtypeStruct((N_NODES, HIDDEN), jnp.float32),
)


def _heads_body(emb_ref, bi_ref, ea_ref, eb_ref,
                ncw_ref, ncb_ref, ecw_ref, ecb_ref, ccw_ref, ccb_ref,
                tcw_ref, tcb_ref, ndw_ref, ndb_ref,
                ee1a_ref, ee1b_ref, eeb1_ref, ee2_ref, eeb2_ref,
                cn1a_ref, cn1b_ref, cnb1_ref, cn2_ref, cnb2_ref,
                sp1a_ref, sp1b_ref, spb1_ref, sp2_ref, spb2_ref,
                nc_ref, ec_ref, cc_ref, tc_ref, nd_ref,
                ee_ref, cn_ref, sp_ref):
    emb = emb_ref[...]
    ohT = (lax.broadcasted_iota(jnp.int32, (N_GRAPHS, N_NODES), 0)
           == bi_ref[...]).astype(jnp.float32)
    pooled = jnp.dot(ohT, emb, preferred_element_type=jnp.float32)
    nc_ref[...] = jnp.dot(pooled, ncw_ref[...], preferred_element_type=jnp.float32) + ncb_ref[...]
    ec_ref[...] = jnp.dot(pooled, ecw_ref[...], preferred_element_type=jnp.float32) + ecb_ref[...]
    cc_ref[...] = jnp.dot(pooled, ccw_ref[...], preferred_element_type=jnp.float32) + ccb_ref[...]
    tc_ref[...] = jnp.dot(pooled, tcw_ref[...], preferred_element_type=jnp.float32) + tcb_ref[...]
    nd_ref[...] = jnp.dot(emb, ndw_ref[...], preferred_element_type=jnp.float32) + ndb_ref[...]

    ea = ea_ref[...]
    eb = eb_ref[...]

    def pair_head(w1a, w1b, b1, w2, b2, out_ref):
        h1 = (jnp.dot(ea, w1a[...], preferred_element_type=jnp.float32)
              + jnp.dot(eb, w1b[...], preferred_element_type=jnp.float32) + b1[...])
        h1 = jnp.maximum(h1, 0.0)
        out_ref[...] = jnp.dot(h1, w2[...], preferred_element_type=jnp.float32) + b2[...]

    pair_head(ee1a_ref, ee1b_ref, eeb1_ref, ee2_ref, eeb2_ref, ee_ref)
    pair_head(cn1a_ref, cn1b_ref, cnb1_ref, cn2_ref, cnb2_ref, cn_ref)
    pair_head(sp1a_ref, sp1b_ref, spb1_ref, sp2_ref, spb2_ref, sp_ref)


_heads_call = pl.pallas_call(
    _heads_body,
    out_shape=[
        jax.ShapeDtypeStruct((N_GRAPHS, 40), jnp.float32),
        jax.ShapeDtypeStruct((N_GRAPHS, 1600), jnp.float32),
        jax.ShapeDtypeStruct((N_GRAPHS, 2), jnp.float32),
        jax.ShapeDtypeStruct((N_GRAPHS, 1), jnp.float32),
        jax.ShapeDtypeStruct((N_NODES, 40), jnp.float32),
        jax.ShapeDtypeStruct((N_PAIRS, 2), jnp.float32),
        jax.ShapeDtypeStruct((N_PAIRS, 2), jnp.float32),
        jax.ShapeDtypeStruct((N_PAIRS, 40), jnp.float32),
    ],
)


def kernel(x, edge_index, batch_index, pairs, params):
    h = x.astype(jnp.bfloat16).astype(jnp.float32)
    src = edge_index[0].astype(jnp.int32).reshape(NW * NCH, CHUNK)
    dst = edge_index[1].astype(jnp.int32).reshape(NW * NCH, CHUNK)
    zeros = jnp.zeros((ROWS_PER_TILE, HIDDEN), jnp.float32)

    row = lambda v: v.reshape(1, -1)
    for i in range(NUM_LAYERS):
        agg = _edge_agg(h, src, dst, zeros)
        h = _layer_call(h, agg,
                        params['conv%d_W1' % i], row(params['conv%d_b1' % i]),
                        params['conv%d_W2' % i], row(params['conv%d_b2' % i]),
                        row(params['bn%d_gamma' % i]), row(params['bn%d_beta' % i]))

    p0 = pairs[:, 0].astype(jnp.int32).reshape(NW, PAIRS_PER_TILE)
    p1 = pairs[:, 1].astype(jnp.int32).reshape(NW, PAIRS_PER_TILE)
    ea, eb = _pair_gather(h, p0, p1)

    def split_w1(name):
        w1 = params[name + '_W1']
        return w1[:HIDDEN], w1[HIDDEN:]

    ee1a, ee1b = split_w1('edge_existence')
    cn1a, cn1b = split_w1('connectivity')
    sp1a, sp1b = split_w1('shortest_path')

    (node_count, edge_count, cycle_check, tri, node_degree,
     edge_existence, connectivity, shortest_path) = _heads_call(
        h, batch_index.astype(jnp.int32).reshape(1, N_NODES), ea, eb,
        params['node_count_W'], row(params['node_count_b']),
        params['edge_count_W'], row(params['edge_count_b']),
        params['cycle_check_W'], row(params['cycle_check_b']),
        params['triangle_count_W'], row(params['triangle_count_b']),
        params['node_degree_W'], row(params['node_degree_b']),
        ee1a, ee1b, row(params['edge_existence_b1']),
        params['edge_existence_W2'], row(params['edge_existence_b2']),
        cn1a, cn1b, row(params['connectivity_b1']),
        params['connectivity_W2'], row(params['connectivity_b2']),
        sp1a, sp1b, row(params['shortest_path_b1']),
        params['shortest_path_W2'], row(params['shortest_path_b2']),
    )
    return (node_count, edge_count, cycle_check, tri[:, 0], node_degree,
            edge_existence, connectivity, shortest_path)


# speedup vs baseline: 7.5383x; 7.5383x over previous
import functools
import jax
import jax.numpy as jnp
from jax import lax
from jax.experimental import pallas as pl
from jax.experimental.pallas import tpu as pltpu
from jax.experimental.pallas import tpu_sc as plsc

N_NODES = 10000
N_EDGES = 320000
HIDDEN = 128
NUM_LAYERS = 3
N_GRAPHS = 64
N_PAIRS = 4096

NC = 2
NS = 16
NW = NC * NS

CHUNK = 80
EDGES_PER_TILE = N_EDGES // NW
NCH = EDGES_PER_TILE // CHUNK
NPAD = 10240
ROWS_PER_TILE = NPAD // NS

_mesh = plsc.VectorSubcoreMesh(core_axis_name="c", subcore_axis_name="s")
_sc_params = pltpu.CompilerParams(use_tc_tiling_on_sc=False)


@functools.partial(
    pl.kernel,
    out_type=jax.ShapeDtypeStruct((NC, NPAD, HIDDEN), jnp.float32),
    mesh=_mesh,
    compiler_params=_sc_params,
    scratch_types=[
        pltpu.VMEM_SHARED((NPAD, HIDDEN), jnp.float32),
        pltpu.VMEM((NCH, CHUNK), jnp.int32),
        pltpu.VMEM((NCH, CHUNK), jnp.int32),
        pltpu.VMEM((CHUNK, HIDDEN), jnp.float32),
        pltpu.VMEM((CHUNK, HIDDEN), jnp.float32),
        pltpu.SemaphoreType.DMA,
        pltpu.SemaphoreType.DMA,
    ],
)
def _edge_agg(h_hbm, src_hbm, dst_hbm, zeros_hbm, out_hbm,
              acc, idx_s, idx_d, rows0, rows1, sem0, sem1):
    c = lax.axis_index("c")
    s = lax.axis_index("s")
    wid = s * NC + c

    pltpu.sync_copy(zeros_hbm, acc.at[pl.ds(s * ROWS_PER_TILE, ROWS_PER_TILE)])
    pltpu.sync_copy(src_hbm.at[wid], idx_s)
    pltpu.sync_copy(dst_hbm.at[wid], idx_d)
    plsc.subcore_barrier()

    pltpu.async_copy(h_hbm.at[idx_s.at[0]], rows0, sem0)

    def body(j, _):
        @pl.when(j % 2 == 0)
        def _even():
            pltpu.make_async_copy(h_hbm.at[idx_s.at[j]], rows0, sem0).wait()

            @pl.when(j + 1 < NCH)
            def _():
                pltpu.async_copy(h_hbm.at[idx_s.at[j + 1]], rows1, sem1)
            pltpu.sync_copy(rows0, acc.at[idx_d.at[j]], add=True)

        @pl.when(j % 2 == 1)
        def _odd():
            pltpu.make_async_copy(h_hbm.at[idx_s.at[j]], rows1, sem1).wait()

            @pl.when(j + 1 < NCH)
            def _():
                pltpu.async_copy(h_hbm.at[idx_s.at[j + 1]], rows0, sem0)
            pltpu.sync_copy(rows1, acc.at[idx_d.at[j]], add=True)
        return _

    lax.fori_loop(0, NCH, body, None)
    plsc.subcore_barrier()

    pltpu.sync_copy(acc.at[pl.ds(s * ROWS_PER_TILE, ROWS_PER_TILE)],
                    out_hbm.at[c].at[pl.ds(s * ROWS_PER_TILE, ROWS_PER_TILE)])


PAIRS_PER_TILE = N_PAIRS // NW


@functools.partial(
    pl.kernel,
    out_type=[
        jax.ShapeDtypeStruct((N_PAIRS, HIDDEN), jnp.float32),
        jax.ShapeDtypeStruct((N_PAIRS, HIDDEN), jnp.float32),
    ],
    mesh=_mesh,
    compiler_params=_sc_params,
    scratch_types=[
        pltpu.VMEM((PAIRS_PER_TILE,), jnp.int32),
        pltpu.VMEM((PAIRS_PER_TILE, HIDDEN), jnp.float32),
        pltpu.SemaphoreType.DMA,
    ],
)
def _pair_gather(emb_hbm, p0_hbm, p1_hbm, ea_hbm, eb_hbm, idx_v, rows_v, sem):
    c = lax.axis_index("c")
    s = lax.axis_index("s")
    wid = s * NC + c
    base = wid * PAIRS_PER_TILE

    pltpu.sync_copy(p0_hbm.at[wid], idx_v)
    pltpu.async_copy(emb_hbm.at[idx_v], rows_v, sem).wait()
    pltpu.sync_copy(rows_v, ea_hbm.at[pl.ds(base, PAIRS_PER_TILE)])

    pltpu.sync_copy(p1_hbm.at[wid], idx_v)
    pltpu.async_copy(emb_hbm.at[idx_v], rows_v, sem).wait()
    pltpu.sync_copy(rows_v, eb_hbm.at[pl.ds(base, PAIRS_PER_TILE)])


def _layer_body(h_ref, a_ref, w1_ref, b1_ref, w2_ref, b2_ref, g_ref, be_ref, o_ref):
    m = h_ref[...] + a_ref[0, :N_NODES] + a_ref[1, :N_NODES]
    z = jnp.dot(m, w1_ref[...], preferred_element_type=jnp.float32) + b1_ref[...]
    z = jnp.maximum(z, 0.0)
    z = jnp.dot(z, w2_ref[...], preferred_element_type=jnp.float32) + b2_ref[...]
    mean = jnp.mean(z, axis=0, keepdims=True)
    cen = z - mean
    var = jnp.mean(cen * cen, axis=0, keepdims=True)
    z = cen * lax.rsqrt(var + 1e-5) * g_ref[...] + be_ref[...]
    o_ref[...] = jnp.maximum(z, 0.0)


_layer_call = pl.pallas_call(
    _layer_body,
    out_shape=jax.ShapeDtypeStruct((N_NODES, HIDDEN), jnp.float32),
)


def _heads_body(emb_ref, bi_ref, ea_ref, eb_ref,
                ncw_ref, ncb_ref, ecw_ref, ecb_ref, ccw_ref, ccb_ref,
                tcw_ref, tcb_ref, ndw_ref, ndb_ref,
                ee1a_ref, ee1b_ref, eeb1_ref, ee2_ref, eeb2_ref,
                cn1a_ref, cn1b_ref, cnb1_ref, cn2_ref, cnb2_ref,
                sp1a_ref, sp1b_ref, spb1_ref, sp2_ref, spb2_ref,
                nc_ref, ec_ref, cc_ref, tc_ref, nd_ref,
                ee_ref, cn_ref, sp_ref):
    emb = emb_ref[...]
    ohT = (lax.broadcasted_iota(jnp.int32, (N_GRAPHS, N_NODES), 0)
           == bi_ref[...]).astype(jnp.float32)
    pooled = jnp.dot(ohT, emb, preferred_element_type=jnp.float32)
    nc_ref[...] = jnp.dot(pooled, ncw_ref[...], preferred_element_type=jnp.float32) + ncb_ref[...]
    ec_ref[...] = jnp.dot(pooled, ecw_ref[...], preferred_element_type=jnp.float32) + ecb_ref[...]
    cc_ref[...] = jnp.dot(pooled, ccw_ref[...], preferred_element_type=jnp.float32) + ccb_ref[...]
    tc_ref[...] = jnp.dot(pooled, tcw_ref[...], preferred_element_type=jnp.float32) + tcb_ref[...]
    nd_ref[...] = jnp.dot(emb, ndw_ref[...], preferred_element_type=jnp.float32) + ndb_ref[...]

    ea = ea_ref[...]
    eb = eb_ref[...]

    def pair_head(w1a, w1b, b1, w2, b2, out_ref):
        h1 = (jnp.dot(ea, w1a[...], preferred_element_type=jnp.float32)
              + jnp.dot(eb, w1b[...], preferred_element_type=jnp.float32) + b1[...])
        h1 = jnp.maximum(h1, 0.0)
        out_ref[...] = jnp.dot(h1, w2[...], preferred_element_type=jnp.float32) + b2[...]

    pair_head(ee1a_ref, ee1b_ref, eeb1_ref, ee2_ref, eeb2_ref, ee_ref)
    pair_head(cn1a_ref, cn1b_ref, cnb1_ref, cn2_ref, cnb2_ref, cn_ref)
    pair_head(sp1a_ref, sp1b_ref, spb1_ref, sp2_ref, spb2_ref, sp_ref)


_heads_call = pl.pallas_call(
    _heads_body,
    out_shape=[
        jax.ShapeDtypeStruct((N_GRAPHS, 40), jnp.float32),
        jax.ShapeDtypeStruct((N_GRAPHS, 1600), jnp.float32),
        jax.ShapeDtypeStruct((N_GRAPHS, 2), jnp.float32),
        jax.ShapeDtypeStruct((N_GRAPHS, 1), jnp.float32),
        jax.ShapeDtypeStruct((N_NODES, 40), jnp.float32),
        jax.ShapeDtypeStruct((N_PAIRS, 2), jnp.float32),
        jax.ShapeDtypeStruct((N_PAIRS, 2), jnp.float32),
        jax.ShapeDtypeStruct((N_PAIRS, 40), jnp.float32),
    ],
)


def kernel(x, edge_index, batch_index, pairs, params):
    h = x.astype(jnp.bfloat16).astype(jnp.float32)
    src = edge_index[0].astype(jnp.int32).reshape(NW, NCH, CHUNK)
    dst = edge_index[1].astype(jnp.int32).reshape(NW, NCH, CHUNK)
    zeros = jnp.zeros((ROWS_PER_TILE, HIDDEN), jnp.float32)

    row = lambda v: v.reshape(1, -1)
    for i in range(NUM_LAYERS):
        agg = _edge_agg(h, src, dst, zeros)
        h = _layer_call(h, agg,
                        params['conv%d_W1' % i], row(params['conv%d_b1' % i]),
                        params['conv%d_W2' % i], row(params['conv%d_b2' % i]),
                        row(params['bn%d_gamma' % i]), row(params['bn%d_beta' % i]))

    p0 = pairs[:, 0].astype(jnp.int32).reshape(NW, PAIRS_PER_TILE)
    p1 = pairs[:, 1].astype(jnp.int32).reshape(NW, PAIRS_PER_TILE)
    ea, eb = _pair_gather(h, p0, p1)

    def split_w1(name):
        w1 = params[name + '_W1']
        return w1[:HIDDEN], w1[HIDDEN:]

    ee1a, ee1b = split_w1('edge_existence')
    cn1a, cn1b = split_w1('connectivity')
    sp1a, sp1b = split_w1('shortest_path')

    (node_count, edge_count, cycle_check, tri, node_degree,
     edge_existence, connectivity, shortest_path) = _heads_call(
        h, batch_index.astype(jnp.int32).reshape(1, N_NODES), ea, eb,
        params['node_count_W'], row(params['node_count_b']),
        params['edge_count_W'], row(params['edge_count_b']),
        params['cycle_check_W'], row(params['cycle_check_b']),
        params['triangle_count_W'], row(params['triangle_count_b']),
        params['node_degree_W'], row(params['node_degree_b']),
        ee1a, ee1b, row(params['edge_existence_b1']),
        params['edge_existence_W2'], row(params['edge_existence_b2']),
        cn1a, cn1b, row(params['connectivity_b1']),
        params['connectivity_W2'], row(params['connectivity_b2']),
        sp1a, sp1b, row(params['shortest_path_b1']),
        params['shortest_path_W2'], row(params['shortest_path_b2']),
    )
    return (node_count, edge_count, cycle_check, tri[:, 0], node_degree,
            edge_existence, connectivity, shortest_path)


# speedup vs baseline: 9.3736x; 1.2435x over previous
import functools
import jax
import jax.numpy as jnp
from jax import lax
from jax.experimental import pallas as pl
from jax.experimental.pallas import tpu as pltpu
from jax.experimental.pallas import tpu_sc as plsc

N_NODES = 10000
N_EDGES = 320000
HIDDEN = 128
NUM_LAYERS = 3
N_GRAPHS = 64
N_PAIRS = 4096

NC = 2
NS = 16
NW = NC * NS

CHUNK = 80
EDGES_PER_TILE = N_EDGES // NW
NCH = EDGES_PER_TILE // CHUNK
NPAD = 10240
ROWS_PER_TILE = NPAD // NS

_mesh = plsc.VectorSubcoreMesh(core_axis_name="c", subcore_axis_name="s")
_sc_params = pltpu.CompilerParams(use_tc_tiling_on_sc=False)


@functools.partial(
    pl.kernel,
    out_type=jax.ShapeDtypeStruct((NC, NPAD, HIDDEN), jnp.float32),
    mesh=_mesh,
    compiler_params=_sc_params,
    scratch_types=[
        pltpu.VMEM_SHARED((NPAD, HIDDEN), jnp.float32),
        pltpu.VMEM((NCH, CHUNK), jnp.int32),
        pltpu.VMEM((NCH, CHUNK), jnp.int32),
        pltpu.VMEM((CHUNK, HIDDEN), jnp.float32),
        pltpu.VMEM((CHUNK, HIDDEN), jnp.float32),
        pltpu.SemaphoreType.DMA,
        pltpu.SemaphoreType.DMA,
        pltpu.SemaphoreType.DMA,
        pltpu.SemaphoreType.DMA,
    ],
)
def _edge_agg(h_hbm, src_hbm, dst_hbm, zeros_hbm, out_hbm,
              acc, idx_s, idx_d, rows0, rows1, gsem0, gsem1, ssem0, ssem1):
    c = lax.axis_index("c")
    s = lax.axis_index("s")
    wid = s * NC + c

    pltpu.sync_copy(zeros_hbm, acc.at[pl.ds(s * ROWS_PER_TILE, ROWS_PER_TILE)])
    pltpu.sync_copy(src_hbm.at[wid], idx_s)
    pltpu.sync_copy(dst_hbm.at[wid], idx_d)
    plsc.subcore_barrier()

    assert NCH % 2 == 1
    pltpu.async_copy(h_hbm.at[idx_s.at[0]], rows0, gsem0)

    def step(j, rows, gsem, ssem, o_rows, o_gsem, o_ssem):
        @pl.when(j > 0)
        def _():
            pltpu.make_async_copy(o_rows, acc.at[idx_d.at[j]], o_ssem).wait()

        @pl.when(j + 1 < NCH)
        def _():
            pltpu.async_copy(h_hbm.at[idx_s.at[j + 1]], o_rows, o_gsem)
        pltpu.make_async_copy(h_hbm.at[idx_s.at[j]], rows, gsem).wait()
        pltpu.async_copy(rows, acc.at[idx_d.at[j]], ssem, add=True)

    def body(j, _):
        @pl.when(j % 2 == 0)
        def _even():
            step(j, rows0, gsem0, ssem0, rows1, gsem1, ssem1)

        @pl.when(j % 2 == 1)
        def _odd():
            step(j, rows1, gsem1, ssem1, rows0, gsem0, ssem0)
        return _

    lax.fori_loop(0, NCH, body, None)
    pltpu.make_async_copy(rows0, acc.at[idx_d.at[NCH - 1]], ssem0).wait()
    plsc.subcore_barrier()

    pltpu.sync_copy(acc.at[pl.ds(s * ROWS_PER_TILE, ROWS_PER_TILE)],
                    out_hbm.at[c].at[pl.ds(s * ROWS_PER_TILE, ROWS_PER_TILE)])


PAIRS_PER_TILE = N_PAIRS // NW


@functools.partial(
    pl.kernel,
    out_type=[
        jax.ShapeDtypeStruct((N_PAIRS, HIDDEN), jnp.float32),
        jax.ShapeDtypeStruct((N_PAIRS, HIDDEN), jnp.float32),
    ],
    mesh=_mesh,
    compiler_params=_sc_params,
    scratch_types=[
        pltpu.VMEM((PAIRS_PER_TILE,), jnp.int32),
        pltpu.VMEM((PAIRS_PER_TILE, HIDDEN), jnp.float32),
        pltpu.SemaphoreType.DMA,
    ],
)
def _pair_gather(emb_hbm, p0_hbm, p1_hbm, ea_hbm, eb_hbm, idx_v, rows_v, sem):
    c = lax.axis_index("c")
    s = lax.axis_index("s")
    wid = s * NC + c
    base = wid * PAIRS_PER_TILE

    pltpu.sync_copy(p0_hbm.at[wid], idx_v)
    pltpu.async_copy(emb_hbm.at[idx_v], rows_v, sem).wait()
    pltpu.sync_copy(rows_v, ea_hbm.at[pl.ds(base, PAIRS_PER_TILE)])

    pltpu.sync_copy(p1_hbm.at[wid], idx_v)
    pltpu.async_copy(emb_hbm.at[idx_v], rows_v, sem).wait()
    pltpu.sync_copy(rows_v, eb_hbm.at[pl.ds(base, PAIRS_PER_TILE)])


def _layer_body(h_ref, a_ref, w1_ref, b1_ref, w2_ref, b2_ref, g_ref, be_ref, o_ref):
    m = h_ref[...] + a_ref[0, :N_NODES] + a_ref[1, :N_NODES]
    z = jnp.dot(m, w1_ref[...], preferred_element_type=jnp.float32) + b1_ref[...]
    z = jnp.maximum(z, 0.0)
    z = jnp.dot(z, w2_ref[...], preferred_element_type=jnp.float32) + b2_ref[...]
    mean = jnp.mean(z, axis=0, keepdims=True)
    cen = z - mean
    var = jnp.mean(cen * cen, axis=0, keepdims=True)
    z = cen * lax.rsqrt(var + 1e-5) * g_ref[...] + be_ref[...]
    o_ref[...] = jnp.maximum(z, 0.0)


_layer_call = pl.pallas_call(
    _layer_body,
    out_shape=jax.ShapeDtypeStruct((N_NODES, HIDDEN), jnp.float32),
)


def _heads_body(emb_ref, bi_ref, ea_ref, eb_ref,
                ncw_ref, ncb_ref, ecw_ref, ecb_ref, ccw_ref, ccb_ref,
                tcw_ref, tcb_ref, ndw_ref, ndb_ref,
                ee1a_ref, ee1b_ref, eeb1_ref, ee2_ref, eeb2_ref,
                cn1a_ref, cn1b_ref, cnb1_ref, cn2_ref, cnb2_ref,
                sp1a_ref, sp1b_ref, spb1_ref, sp2_ref, spb2_ref,
                nc_ref, ec_ref, cc_ref, tc_ref, nd_ref,
                ee_ref, cn_ref, sp_ref):
    emb = emb_ref[...]
    ohT = (lax.broadcasted_iota(jnp.int32, (N_GRAPHS, N_NODES), 0)
           == bi_ref[...]).astype(jnp.float32)
    pooled = jnp.dot(ohT, emb, preferred_element_type=jnp.float32)
    nc_ref[...] = jnp.dot(pooled, ncw_ref[...], preferred_element_type=jnp.float32) + ncb_ref[...]
    ec_ref[...] = jnp.dot(pooled, ecw_ref[...], preferred_element_type=jnp.float32) + ecb_ref[...]
    cc_ref[...] = jnp.dot(pooled, ccw_ref[...], preferred_element_type=jnp.float32) + ccb_ref[...]
    tc_ref[...] = jnp.dot(pooled, tcw_ref[...], preferred_element_type=jnp.float32) + tcb_ref[...]
    nd_ref[...] = jnp.dot(emb, ndw_ref[...], preferred_element_type=jnp.float32) + ndb_ref[...]

    ea = ea_ref[...]
    eb = eb_ref[...]

    def pair_head(w1a, w1b, b1, w2, b2, out_ref):
        h1 = (jnp.dot(ea, w1a[...], preferred_element_type=jnp.float32)
              + jnp.dot(eb, w1b[...], preferred_element_type=jnp.float32) + b1[...])
        h1 = jnp.maximum(h1, 0.0)
        out_ref[...] = jnp.dot(h1, w2[...], preferred_element_type=jnp.float32) + b2[...]

    pair_head(ee1a_ref, ee1b_ref, eeb1_ref, ee2_ref, eeb2_ref, ee_ref)
    pair_head(cn1a_ref, cn1b_ref, cnb1_ref, cn2_ref, cnb2_ref, cn_ref)
    pair_head(sp1a_ref, sp1b_ref, spb1_ref, sp2_ref, spb2_ref, sp_ref)


_heads_call = pl.pallas_call(
    _heads_body,
    out_shape=[
        jax.ShapeDtypeStruct((N_GRAPHS, 40), jnp.float32),
        jax.ShapeDtypeStruct((N_GRAPHS, 1600), jnp.float32),
        jax.ShapeDtypeStruct((N_GRAPHS, 2), jnp.float32),
        jax.ShapeDtypeStruct((N_GRAPHS, 1), jnp.float32),
        jax.ShapeDtypeStruct((N_NODES, 40), jnp.float32),
        jax.ShapeDtypeStruct((N_PAIRS, 2), jnp.float32),
        jax.ShapeDtypeStruct((N_PAIRS, 2), jnp.float32),
        jax.ShapeDtypeStruct((N_PAIRS, 40), jnp.float32),
    ],
)


def kernel(x, edge_index, batch_index, pairs, params):
    h = x.astype(jnp.bfloat16).astype(jnp.float32)
    src = edge_index[0].astype(jnp.int32).reshape(NW, NCH, CHUNK)
    dst = edge_index[1].astype(jnp.int32).reshape(NW, NCH, CHUNK)
    zeros = jnp.zeros((ROWS_PER_TILE, HIDDEN), jnp.float32)

    row = lambda v: v.reshape(1, -1)
    for i in range(NUM_LAYERS):
        agg = _edge_agg(h, src, dst, zeros)
        h = _layer_call(h, agg,
                        params['conv%d_W1' % i], row(params['conv%d_b1' % i]),
                        params['conv%d_W2' % i], row(params['conv%d_b2' % i]),
                        row(params['bn%d_gamma' % i]), row(params['bn%d_beta' % i]))

    p0 = pairs[:, 0].astype(jnp.int32).reshape(NW, PAIRS_PER_TILE)
    p1 = pairs[:, 1].astype(jnp.int32).reshape(NW, PAIRS_PER_TILE)
    ea, eb = _pair_gather(h, p0, p1)

    def split_w1(name):
        w1 = params[name + '_W1']
        return w1[:HIDDEN], w1[HIDDEN:]

    ee1a, ee1b = split_w1('edge_existence')
    cn1a, cn1b = split_w1('connectivity')
    sp1a, sp1b = split_w1('shortest_path')

    (node_count, edge_count, cycle_check, tri, node_degree,
     edge_existence, connectivity, shortest_path) = _heads_call(
        h, batch_index.astype(jnp.int32).reshape(1, N_NODES), ea, eb,
        params['node_count_W'], row(params['node_count_b']),
        params['edge_count_W'], row(params['edge_count_b']),
        params['cycle_check_W'], row(params['cycle_check_b']),
        params['triangle_count_W'], row(params['triangle_count_b']),
        params['node_degree_W'], row(params['node_degree_b']),
        ee1a, ee1b, row(params['edge_existence_b1']),
        params['edge_existence_W2'], row(params['edge_existence_b2']),
        cn1a, cn1b, row(params['connectivity_b1']),
        params['connectivity_W2'], row(params['connectivity_b2']),
        sp1a, sp1b, row(params['shortest_path_b1']),
        params['shortest_path_W2'], row(params['shortest_path_b2']),
    )
    return (node_count, edge_count, cycle_check, tri[:, 0], node_degree,
            edge_existence, connectivity, shortest_path)
